# baseline (speedup 1.0000x reference)
import jax
import jax.numpy as jnp
from jax.experimental import pallas as pl
from jax.experimental.pallas import tpu as pltpu

_N_UNITS = 4096
_K = 410
_BOOST_STRENGTH = 1.0
_TARGET_DENSITY = float(_K) / _N_UNITS
_ROWS = 512


def _count_cols(s, ones_bf16):
    w = s.shape[1]
    while w > 128:
        w //= 2
        s = s[:, :w] + s[:, w:]
    c = jax.lax.dot_general(
        s.astype(jnp.bfloat16), ones_bf16,
        (((1,), (0,)), ((), ())),
        preferred_element_type=jnp.float32,
    )
    return c[:, :1]


def _bisect16(vals16, quota_f, ones_bf16, n_chunks=8):
    R = vals16.shape[0]
    rc = R // n_chunks
    chunks = [vals16[i * rc:(i + 1) * rc] for i in range(n_chunks)]
    quotas = [quota_f[i * rc:(i + 1) * rc] for i in range(n_chunks)]
    Ts = [jnp.zeros((rc, 1), jnp.int32) for _ in range(n_chunks)]
    for b in range(15, -1, -1):
        bit = jnp.int32(1 << b)
        trials = [T | bit for T in Ts]
        for i in range(n_chunks):
            thr16 = (trials[i] - jnp.int32(0x8000)).astype(jnp.int16)
            s = (chunks[i] >= thr16).astype(jnp.int16)
            cnt = _count_cols(s, ones_bf16)
            Ts[i] = jnp.where(cnt >= quotas[i], trials[i], Ts[i])
    return jnp.concatenate(Ts, axis=0) - jnp.int32(0x8000)


def _kwinners_block(x_ref, dc_ref, o_ref):
    xb = x_ref[...]
    dc = dc_ref[...]
    bf = jnp.exp((_TARGET_DENSITY - dc) * _BOOST_STRENGTH)
    boosted = xb * bf

    t = jax.lax.bitcast_convert_type(boosted, jnp.int32)
    key = t ^ ((t >> 31) & jnp.int32(0x7FFFFFFF))

    R = xb.shape[0]
    ones_bf16 = jnp.ones((128, 128), jnp.bfloat16)

    k_hi = (key >> 16).astype(jnp.int16)
    k_lo = ((key & jnp.int32(0xFFFF)) - jnp.int32(0x8000)).astype(jnp.int16)

    quota_f = jnp.full((R, 1), float(_K), jnp.float32)

    t_hi = _bisect16(k_hi, quota_f, ones_bf16)
    t_hi16 = t_hi.astype(jnp.int16)

    cnt_gt = _count_cols((k_hi > t_hi16).astype(jnp.int16), ones_bf16)
    k_rem_f = quota_f - cnt_gt

    eq = k_hi == t_hi16
    ml = jnp.where(eq, k_lo, jnp.int16(-32768))

    t_lo = _bisect16(ml, k_rem_f, ones_bf16)

    kth = (t_hi << 16) | (t_lo + jnp.int32(0x8000))
    o_ref[...] = jnp.where(key >= kth, xb, 0.0)


def kernel(x, dutyCycle):
    B, N = x.shape
    dc = dutyCycle.reshape(1, N)
    return pl.pallas_call(
        _kwinners_block,
        grid=(B // _ROWS,),
        in_specs=[
            pl.BlockSpec((_ROWS, N), lambda i: (i, 0)),
            pl.BlockSpec((1, N), lambda i: (0, 0)),
        ],
        out_specs=pl.BlockSpec((_ROWS, N), lambda i: (i, 0)),
        out_shape=jax.ShapeDtypeStruct((B, N), x.dtype),
        compiler_params=pltpu.CompilerParams(
            dimension_semantics=("arbitrary",),
        ),
    )(x, dc)


# speedup vs baseline: 1.0008x; 1.0008x over previous
import jax
import jax.numpy as jnp
from jax.experimental import pallas as pl
from jax.experimental.pallas import tpu as pltpu

_N_UNITS = 4096
_K = 410
_BOOST_STRENGTH = 1.0
_TARGET_DENSITY = float(_K) / _N_UNITS
_ROWS = 512


def _count_cols(s, ones_bf16):
    w = s.shape[1]
    while w > 128:
        w //= 2
        s = s[:, :w] + s[:, w:]
    c = jax.lax.dot_general(
        s.astype(jnp.bfloat16), ones_bf16,
        (((1,), (0,)), ((), ())),
        preferred_element_type=jnp.float32,
    )
    return c[:, :1]


def _bisect16(vals16, quota_f, ones_bf16, n_chunks=4):
    R = vals16.shape[0]
    rc = R // n_chunks
    chunks = [vals16[i * rc:(i + 1) * rc] for i in range(n_chunks)]
    quotas = [quota_f[i * rc:(i + 1) * rc] for i in range(n_chunks)]
    Ts = [jnp.zeros((rc, 1), jnp.int32) for _ in range(n_chunks)]
    for b in range(15, -1, -1):
        bit = jnp.int32(1 << b)
        trials = [T | bit for T in Ts]
        for i in range(n_chunks):
            thr16 = (trials[i] - jnp.int32(0x8000)).astype(jnp.int16)
            s = (chunks[i] >= thr16).astype(jnp.int16)
            cnt = _count_cols(s, ones_bf16)
            Ts[i] = jnp.where(cnt >= quotas[i], trials[i], Ts[i])
    return jnp.concatenate(Ts, axis=0) - jnp.int32(0x8000)


def _kwinners_block(x_ref, dc_ref, o_ref):
    xb = x_ref[...]
    dc = dc_ref[...]
    bf = jnp.exp((_TARGET_DENSITY - dc) * _BOOST_STRENGTH)
    boosted = xb * bf

    t = jax.lax.bitcast_convert_type(boosted, jnp.int32)
    key = t ^ ((t >> 31) & jnp.int32(0x7FFFFFFF))

    R = xb.shape[0]
    ones_bf16 = jnp.ones((128, 128), jnp.bfloat16)

    k_hi = (key >> 16).astype(jnp.int16)
    k_lo = ((key & jnp.int32(0xFFFF)) - jnp.int32(0x8000)).astype(jnp.int16)

    quota_f = jnp.full((R, 1), float(_K), jnp.float32)

    t_hi = _bisect16(k_hi, quota_f, ones_bf16)
    t_hi16 = t_hi.astype(jnp.int16)

    cnt_gt = _count_cols((k_hi > t_hi16).astype(jnp.int16), ones_bf16)
    k_rem_f = quota_f - cnt_gt

    eq = k_hi == t_hi16
    ml = jnp.where(eq, k_lo, jnp.int16(-32768))

    t_lo = _bisect16(ml, k_rem_f, ones_bf16)

    kth = (t_hi << 16) | (t_lo + jnp.int32(0x8000))
    o_ref[...] = jnp.where(key >= kth, xb, 0.0)


def kernel(x, dutyCycle):
    B, N = x.shape
    dc = dutyCycle.reshape(1, N)
    return pl.pallas_call(
        _kwinners_block,
        grid=(B // _ROWS,),
        in_specs=[
            pl.BlockSpec((_ROWS, N), lambda i: (i, 0)),
            pl.BlockSpec((1, N), lambda i: (0, 0)),
        ],
        out_specs=pl.BlockSpec((_ROWS, N), lambda i: (i, 0)),
        out_shape=jax.ShapeDtypeStruct((B, N), x.dtype),
        compiler_params=pltpu.CompilerParams(
            dimension_semantics=("arbitrary",),
        ),
    )(x, dc)
